# baseline (speedup 1.0000x reference)
import functools
import math

import jax
import jax.numpy as jnp
import numpy as np
from jax import lax
from jax.experimental import pallas as pl
from jax.experimental.pallas import tpu as pltpu
from jax.experimental.pallas import tpu_sc as plsc

_BS = 2
_NQ = 5440
_EMBED = 256
_HEADS = 8
_LEVELS = 4
_POINTS = 4
_DH = _EMBED // _HEADS
_HLP = _HEADS * _LEVELS * _POINTS
_QB = 680
_NQB = _NQ // _QB

_SIZES = [64 >> l for l in range(_LEVELS)]
_BASES = [0, 4096, 5120, 5376]


def _prep_body(q_ref, rx_ref, ry_ref, wval_ref, bval_ref, wox_ref, box_ref,
               woy_ref, boy_ref, wattn_ref, battn_ref, e_ref, g_ref,
               val_ref, idx_ref, wgt_ref):
    b = pl.program_id(0)
    q = q_ref[0]
    val = (jnp.dot(q, wval_ref[...], preferred_element_type=jnp.float32, precision=jax.lax.Precision.HIGHEST)
           + bval_ref[...])
    for hh in range(_HEADS):
        val_ref[0, hh] = val[:, hh * _DH:(hh + 1) * _DH]

    offx = jnp.dot(q, wox_ref[...], preferred_element_type=jnp.float32, precision=jax.lax.Precision.HIGHEST) + box_ref[...]
    offy = jnp.dot(q, woy_ref[...], preferred_element_type=jnp.float32, precision=jax.lax.Precision.HIGHEST) + boy_ref[...]
    aw = jnp.dot(q, wattn_ref[...], preferred_element_type=jnp.float32, precision=jax.lax.Precision.HIGHEST) + battn_ref[...]
    m = jnp.max(aw, axis=-1, keepdims=True)
    ex = jnp.exp(aw - m)
    a = ex / jnp.dot(ex, g_ref[...], preferred_element_type=jnp.float32, precision=jax.lax.Precision.HIGHEST)

    col = lax.broadcasted_iota(jnp.int32, (1, _HLP), 1)
    lvl = (col >> 2) & 3
    head = col >> 4
    wi = jnp.right_shift(jnp.int32(64), lvl)
    wf = wi.astype(jnp.float32)
    base = jnp.where(lvl == 0, 0,
                     jnp.where(lvl == 1, _BASES[1],
                               jnp.where(lvl == 2, _BASES[2], _BASES[3])))

    rx = jnp.dot(rx_ref[0], e_ref[...], preferred_element_type=jnp.float32, precision=jax.lax.Precision.HIGHEST)
    ry = jnp.dot(ry_ref[0], e_ref[...], preferred_element_type=jnp.float32, precision=jax.lax.Precision.HIGHEST)
    x = rx * wf + offx - 0.5
    y = ry * wf + offy - 0.5

    x0f = jnp.floor(x)
    y0f = jnp.floor(y)
    fx = x - x0f
    fy = y - y0f
    x0 = x0f.astype(jnp.int32)
    y0 = y0f.astype(jnp.int32)
    x1 = x0 + 1
    y1 = y0 + 1
    vx0 = (x0 >= 0) & (x0 < wi)
    vx1 = (x1 >= 0) & (x1 < wi)
    vy0 = (y0 >= 0) & (y0 < wi)
    vy1 = (y1 >= 0) & (y1 < wi)
    xc0 = jnp.clip(x0, 0, wi - 1)
    xc1 = jnp.clip(x1, 0, wi - 1)
    yc0 = jnp.clip(y0, 0, wi - 1)
    yc1 = jnp.clip(y1, 0, wi - 1)

    xs = jnp.clip(x0, 0, wi - 2)
    gx0 = 1.0 - fx
    gy0 = 1.0 - fy
    f32 = lambda c: c.astype(jnp.float32)
    wax = gx0 * f32(x0 == xs) + fx * f32(x1 == xs)
    wbx = gx0 * f32(x0 == xs + 1) + fx * f32(x1 == xs + 1)
    wtop = gy0 * f32(vy0) * a
    wbot = fy * f32(vy1) * a
    wat = wax * wtop
    wab = wax * wbot
    wbt = wbx * wtop
    wbb = wbx * wbot
    qoff = (b * _HEADS + head) * _NQ + base
    itop = qoff + yc0 * wi + xs
    ibot = qoff + yc1 * wi + xs

    for hh in range(_HEADS):
        sl = slice(hh * 16, hh * 16 + 16)
        idx_ref[hh] = jnp.concatenate([itop[:, sl], ibot[:, sl]], axis=-1)
        wgt_ref[hh] = jnp.concatenate(
            [wat[:, sl], wab[:, sl], wbt[:, sl], wbb[:, sl]], axis=-1)


def _pair_body(va_ref, vb_ref, tab_ref):
    va = va_ref[0, 0]
    vb = vb_ref[0, 0]
    shifted = jnp.concatenate([va[1:], vb[:1]], axis=0)
    tab_ref[0, 0] = jnp.concatenate([va, shifted], axis=-1)


def _outp_body(s_ref, q_ref, wout_ref, bout_ref, o_ref):
    o_ref[0] = (jnp.dot(s_ref[0], wout_ref[...], preferred_element_type=jnp.float32, precision=jax.lax.Precision.HIGHEST)
                + bout_ref[...] + q_ref[0])


_NC = 2
_NS = 16
_QG = 16
_NG = (_NQ // 2) // _QG


_NGT = _NQ // _QG


def _sample_body(vtab, idx_hbm, wgt_hbm, out, idxb, wgtb, rows, outb,
                 sem_iw, sem_r0, sem_r1):
    wid = lax.axis_index("s") * _NC + lax.axis_index("c")
    b = wid >> 4
    t = wid & 15
    ng = (_NGT - t + 15) // 16
    lane = lax.iota(jnp.int32, _QG)
    jvec = lane >> 1
    pbase = (lane & 1) * 64

    def issue_iw(gslot, g):
        pltpu.async_copy(idx_hbm.at[pl.ds(0, _HEADS), b, g], idxb.at[gslot], sem_iw)
        pltpu.async_copy(wgt_hbm.at[pl.ds(0, _HEADS), b, g], wgtb.at[gslot], sem_iw)

    def wait_iw(gslot):
        pltpu.make_async_copy(
            idx_hbm.at[pl.ds(0, _HEADS), b, 0], idxb.at[gslot], sem_iw).wait()
        pltpu.make_async_copy(
            wgt_hbm.at[pl.ds(0, _HEADS), b, 0], wgtb.at[gslot], sem_iw).wait()

    def issue_gathers(gslot, h, rslot, sem):
        for j in range(4):
            pltpu.async_copy(vtab.at[idxb.at[gslot, h, j]], rows.at[rslot, j], sem)

    def wait_gathers(gslot, h, rslot, sem):
        for j in range(4):
            pltpu.make_async_copy(
                vtab.at[idxb.at[gslot, h, j]], rows.at[rslot, j], sem).wait()

    issue_iw(0, t)
    wait_iw(0)
    issue_gathers(0, 0, 0, sem_r0)
    issue_iw(1, t + 16)

    def group(i, carry):
        s = i & 1
        g = t + i * 16
        for h in range(_HEADS):
            rs = h & 1
            sem_cur = sem_r1 if rs else sem_r0
            wait_gathers(s, h, rs, sem_cur)
            if h < _HEADS - 1:
                nrs = (h + 1) & 1
                issue_gathers(s, h + 1, nrs, sem_r1 if nrs else sem_r0)

            def qstep(q, carry, h=h, rs=rs, s=s):
                wslab = q >> 1
                wpb = (q & 1) * 64
                rslab = q >> 2
                rpb = (q & 3) * 32
                acc0 = jnp.zeros((_QG,), jnp.float32)
                acc1 = jnp.zeros((_QG,), jnp.float32)

                def splat(vec, u):
                    return lax.gather(
                        vec, jnp.full((_QG, 1), u, jnp.int32),
                        dimension_numbers=lax.GatherDimensionNumbers(
                            offset_dims=(), collapsed_slice_dims=(0,),
                            start_index_map=(0,)),
                        slice_sizes=(1,),
                        mode=lax.GatherScatterMode.PROMISE_IN_BOUNDS)

                for kb in range(2):
                    wva = wgtb[s, h, wslab, pl.ds(wpb + kb * 16, 16)]
                    wvb = wgtb[s, h, wslab, pl.ds(wpb + 32 + kb * 16, 16)]
                    for u in range(16):
                        pos = rpb + kb * 16 + u
                        wa = splat(wva, u)
                        wb = splat(wvb, u)
                        va0 = rows[rs, rslab, pos, pl.ds(0, 16)]
                        va1 = rows[rs, rslab, pos, pl.ds(16, 16)]
                        vb0 = rows[rs, rslab, pos, pl.ds(32, 16)]
                        vb1 = rows[rs, rslab, pos, pl.ds(48, 16)]
                        acc0 = acc0 + wa * va0 + wb * vb0
                        acc1 = acc1 + wa * va1 + wb * vb1
                outb[q, pl.ds(h * _DH, 16)] = acc0
                outb[q, pl.ds(h * _DH + 16, 16)] = acc1
                return carry

            lax.fori_loop(0, _QG, qstep, 0)

        @pl.when(i < ng - 1)
        def _():
            wait_iw((i + 1) & 1)
            issue_gathers((i + 1) & 1, 0, 0, sem_r0)

        @pl.when(i < ng - 2)
        def _():
            issue_iw(s, g + 32)

        pltpu.sync_copy(outb, out.at[b, pl.ds(g * _QG, _QG), pl.ds(0, _EMBED)])
        return carry

    lax.fori_loop(0, ng, group, 0)


def _make_sample():
    mesh = plsc.VectorSubcoreMesh(core_axis_name="c", subcore_axis_name="s")
    return functools.partial(
        pl.kernel,
        mesh=mesh,
        compiler_params=pltpu.CompilerParams(
            needs_layout_passes=False, use_tc_tiling_on_sc=False),
        out_type=jax.ShapeDtypeStruct((_BS, _NQ, _EMBED), jnp.float32),
        scratch_types=[
            pltpu.VMEM((2, _HEADS, 4, 128), jnp.int32),
            pltpu.VMEM((2, _HEADS, 8, 128), jnp.float32),
            pltpu.VMEM((2, 4, 128, 2 * _DH), jnp.float32),
            pltpu.VMEM((_QG, _EMBED), jnp.float32),
            pltpu.SemaphoreType.DMA,
            pltpu.SemaphoreType.DMA,
            pltpu.SemaphoreType.DMA,
        ],
    )(_sample_body)


_E_NP = np.zeros((_LEVELS, _HLP), np.float32)
for _j in range(_HLP):
    _E_NP[(_j >> 2) & 3, _j] = 1.0
_G_NP = np.zeros((_HLP, _HLP), np.float32)
for _k in range(_HLP):
    for _j in range(_HLP):
        if (_k >> 4) == (_j >> 4):
            _G_NP[_k, _j] = 1.0


def kernel(query, reference_points, spatial_shapes, level_start_index,
           W_off, b_off, W_attn, b_attn, W_val, b_val, W_out, b_out):
    del spatial_shapes, level_start_index
    rx = reference_points[..., 0]
    ry = reference_points[..., 1]
    wox = W_off.reshape(_EMBED, _HLP, 2)[..., 0]
    woy = W_off.reshape(_EMBED, _HLP, 2)[..., 1]
    box = b_off.reshape(_HLP, 2)[:, 0].reshape(1, _HLP)
    boy = b_off.reshape(_HLP, 2)[:, 1].reshape(1, _HLP)
    bval = b_val.reshape(1, _EMBED)
    battn = b_attn.reshape(1, _HLP)
    bout = b_out.reshape(1, _EMBED)
    e_mat = jnp.asarray(_E_NP)
    g_mat = jnp.asarray(_G_NP)

    full = lambda shape: pl.BlockSpec(shape, lambda b_, q_: tuple(0 for _ in shape))
    val, idx, wgt = pl.pallas_call(
        _prep_body,
        grid=(_BS, _NQB),
        in_specs=[
            pl.BlockSpec((1, _QB, _EMBED), lambda b_, q_: (b_, q_, 0)),
            pl.BlockSpec((1, _QB, _LEVELS), lambda b_, q_: (b_, q_, 0)),
            pl.BlockSpec((1, _QB, _LEVELS), lambda b_, q_: (b_, q_, 0)),
            full((_EMBED, _EMBED)),
            full((1, _EMBED)),
            full((_EMBED, _HLP)),
            full((1, _HLP)),
            full((_EMBED, _HLP)),
            full((1, _HLP)),
            full((_EMBED, _HLP)),
            full((1, _HLP)),
            full((_LEVELS, _HLP)),
            full((_HLP, _HLP)),
        ],
        out_specs=[
            pl.BlockSpec((1, _HEADS, _QB, _DH), lambda b_, q_: (b_, 0, q_, 0)),
            pl.BlockSpec((_HEADS, _QB, 32), lambda b_, q_: (0, b_ * _NQB + q_, 0)),
            pl.BlockSpec((_HEADS, _QB, 64), lambda b_, q_: (0, b_ * _NQB + q_, 0)),
        ],
        out_shape=[
            jax.ShapeDtypeStruct((_BS, _HEADS, _NQ, _DH), jnp.float32),
            jax.ShapeDtypeStruct((_HEADS, _BS * _NQ, 32), jnp.int32),
            jax.ShapeDtypeStruct((_HEADS, _BS * _NQ, 64), jnp.float32),
        ],
    )(query, rx, ry, W_val, bval, wox, box, woy, boy, W_attn, battn, e_mat, g_mat)

    tab = pl.pallas_call(
        _pair_body,
        grid=(_BS, _HEADS, _NQB),
        in_specs=[
            pl.BlockSpec((1, 1, _QB, _DH), lambda b_, h_, q_: (b_, h_, q_, 0)),
            pl.BlockSpec((1, 1, _QB, _DH),
                         lambda b_, h_, q_: (b_, h_, jnp.minimum(q_ + 1, _NQB - 1), 0)),
        ],
        out_specs=pl.BlockSpec((1, 1, _QB, 2 * _DH),
                               lambda b_, h_, q_: (b_, h_, q_, 0)),
        out_shape=jax.ShapeDtypeStruct((_BS, _HEADS, _NQ, 2 * _DH), jnp.float32),
    )(val, val)

    vtab = tab.reshape(_BS * _HEADS * _NQ, 2 * _DH)
    idx5 = idx.reshape(_HEADS, _BS, _NQ // _QG, 4, 128)
    wgt5 = wgt.reshape(_HEADS, _BS, _NQ // _QG, 8, 128)
    sampled = _make_sample()(vtab, idx5, wgt5)

    out = pl.pallas_call(
        _outp_body,
        grid=(_BS, _NQB),
        in_specs=[
            pl.BlockSpec((1, _QB, _EMBED), lambda b_, q_: (b_, q_, 0)),
            pl.BlockSpec((1, _QB, _EMBED), lambda b_, q_: (b_, q_, 0)),
            full((_EMBED, _EMBED)),
            full((1, _EMBED)),
        ],
        out_specs=pl.BlockSpec((1, _QB, _EMBED), lambda b_, q_: (b_, q_, 0)),
        out_shape=jax.ShapeDtypeStruct((_BS, _NQ, _EMBED), jnp.float32),
    )(sampled, query, W_out, bout)
    return out


# speedup vs baseline: 1.4667x; 1.4667x over previous
import functools
import math

import jax
import jax.numpy as jnp
import numpy as np
from jax import lax
from jax.experimental import pallas as pl
from jax.experimental.pallas import tpu as pltpu
from jax.experimental.pallas import tpu_sc as plsc

_BS = 2
_NQ = 5440
_EMBED = 256
_HEADS = 8
_LEVELS = 4
_POINTS = 4
_DH = _EMBED // _HEADS
_HLP = _HEADS * _LEVELS * _POINTS
_QB = 544
_NQB = _NQ // _QB

_SIZES = [64 >> l for l in range(_LEVELS)]
_BASES = [0, 4096, 5120, 5376]


def _prep_body(q_ref, rx_ref, ry_ref, wval_ref, bval_ref, wox_ref, box_ref,
               woy_ref, boy_ref, wattn_ref, battn_ref, e_ref, g_ref,
               val_ref, idx_ref, wgt_ref):
    b = pl.program_id(0)
    q = q_ref[0]
    val_ref[0] = (jnp.dot(q, wval_ref[...], preferred_element_type=jnp.float32, precision=jax.lax.Precision.HIGHEST)
                  + bval_ref[...]).astype(jnp.bfloat16)

    offx = jnp.dot(q, wox_ref[...], preferred_element_type=jnp.float32, precision=jax.lax.Precision.HIGHEST) + box_ref[...]
    offy = jnp.dot(q, woy_ref[...], preferred_element_type=jnp.float32, precision=jax.lax.Precision.HIGHEST) + boy_ref[...]
    aw = jnp.dot(q, wattn_ref[...], preferred_element_type=jnp.float32, precision=jax.lax.Precision.HIGHEST) + battn_ref[...]
    m = jnp.max(aw, axis=-1, keepdims=True)
    ex = jnp.exp(aw - m)
    a = ex / jnp.dot(ex, g_ref[...], preferred_element_type=jnp.float32, precision=jax.lax.Precision.HIGHEST)

    col = lax.broadcasted_iota(jnp.int32, (1, _HLP), 1)
    lvl = (col >> 2) & 3
    head = col >> 4
    wi = jnp.right_shift(jnp.int32(64), lvl)
    wf = wi.astype(jnp.float32)
    base = jnp.where(lvl == 0, 0,
                     jnp.where(lvl == 1, _BASES[1],
                               jnp.where(lvl == 2, _BASES[2], _BASES[3])))

    rx = jnp.dot(rx_ref[0], e_ref[...], preferred_element_type=jnp.float32, precision=jax.lax.Precision.HIGHEST)
    ry = jnp.dot(ry_ref[0], e_ref[...], preferred_element_type=jnp.float32, precision=jax.lax.Precision.HIGHEST)
    x = rx * wf + offx - 0.5
    y = ry * wf + offy - 0.5

    x0f = jnp.floor(x)
    y0f = jnp.floor(y)
    fx = x - x0f
    fy = y - y0f
    x0 = x0f.astype(jnp.int32)
    y0 = y0f.astype(jnp.int32)
    x1 = x0 + 1
    y1 = y0 + 1
    vx0 = (x0 >= 0) & (x0 < wi)
    vx1 = (x1 >= 0) & (x1 < wi)
    vy0 = (y0 >= 0) & (y0 < wi)
    vy1 = (y1 >= 0) & (y1 < wi)
    xc0 = jnp.clip(x0, 0, wi - 1)
    xc1 = jnp.clip(x1, 0, wi - 1)
    yc0 = jnp.clip(y0, 0, wi - 1)
    yc1 = jnp.clip(y1, 0, wi - 1)

    qoff = (b * _NQ + base) * _HEADS + head

    def flat(yc, xc):
        return qoff + (yc * wi + xc) * _HEADS

    gx0 = 1.0 - fx
    gy0 = 1.0 - fy
    i00 = flat(yc0, xc0)
    i10 = flat(yc0, xc1)
    i01 = flat(yc1, xc0)
    i11 = flat(yc1, xc1)
    w00 = gx0 * gy0 * (vx0 & vy0).astype(jnp.float32) * a
    w10 = fx * gy0 * (vx1 & vy0).astype(jnp.float32) * a
    w01 = gx0 * fy * (vx0 & vy1).astype(jnp.float32) * a
    w11 = fx * fy * (vx1 & vy1).astype(jnp.float32) * a

    for hh in range(_HEADS):
        sl = slice(hh * 16, hh * 16 + 16)
        idx_ref[hh] = jnp.concatenate(
            [i00[:, sl], i10[:, sl], i01[:, sl], i11[:, sl]], axis=-1)
        wgt_ref[hh] = jnp.concatenate(
            [w00[:, sl], w10[:, sl], w01[:, sl], w11[:, sl]], axis=-1)


def _outp_body(s_ref, q_ref, wout_ref, bout_ref, o_ref):
    o_ref[0] = (jnp.dot(s_ref[0], wout_ref[...], preferred_element_type=jnp.float32, precision=jax.lax.Precision.HIGHEST)
                + bout_ref[...] + q_ref[0])


_NC = 2
_NS = 16
_QG = 16
_NG = (_NQ // 2) // _QG


_NGT = _NQ // _QG


def _sample_body(vtab, idx_hbm, wgt_hbm, out, idxb, wgtb, rows, outb,
                 sem_iw, sem_r0, sem_r1):
    wid = lax.axis_index("s") * _NC + lax.axis_index("c")
    b = wid >> 4
    t = wid & 15
    ng = (_NGT - t + 15) // 16
    lane = lax.iota(jnp.int32, _QG)
    jvec = lane >> 1
    pbase = (lane & 1) * 64

    def issue_iw(gslot, g):
        pltpu.async_copy(idx_hbm.at[pl.ds(0, _HEADS), b, g], idxb.at[gslot], sem_iw)
        pltpu.async_copy(wgt_hbm.at[pl.ds(0, _HEADS), b, g], wgtb.at[gslot], sem_iw)

    def wait_iw(gslot):
        pltpu.make_async_copy(
            idx_hbm.at[pl.ds(0, _HEADS), b, 0], idxb.at[gslot], sem_iw).wait()
        pltpu.make_async_copy(
            wgt_hbm.at[pl.ds(0, _HEADS), b, 0], wgtb.at[gslot], sem_iw).wait()

    def issue_gathers(gslot, h, rslot, sem):
        for j in range(8):
            pltpu.async_copy(vtab.at[idxb.at[gslot, h, j]], rows.at[rslot, j], sem)

    def wait_gathers(gslot, h, rslot, sem):
        for j in range(8):
            pltpu.make_async_copy(
                vtab.at[idxb.at[gslot, h, j]], rows.at[rslot, j], sem).wait()

    issue_iw(0, t)
    wait_iw(0)
    issue_gathers(0, 0, 0, sem_r0)
    issue_iw(1, t + 16)

    def group(i, carry):
        s = i & 1
        g = t + i * 16
        for h in range(_HEADS):
            rs = h & 1
            sem_cur = sem_r1 if rs else sem_r0
            wait_gathers(s, h, rs, sem_cur)
            if h < _HEADS - 1:
                nrs = (h + 1) & 1
                issue_gathers(s, h + 1, nrs, sem_r1 if nrs else sem_r0)

            def qstep(q, carry, h=h, rs=rs, s=s):
                slab = q >> 1
                pb = (q & 1) * 64
                acc0 = jnp.zeros((_QG,), jnp.float32)
                acc1 = jnp.zeros((_QG,), jnp.float32)
                for k in range(4):
                    base = pb + k * 16
                    wvec = wgtb[s, h, slab, pl.ds(base, 16)]
                    for u in range(16):
                        wj = lax.gather(
                            wvec, jnp.full((_QG, 1), u, jnp.int32),
                            dimension_numbers=lax.GatherDimensionNumbers(
                                offset_dims=(), collapsed_slice_dims=(0,),
                                start_index_map=(0,)),
                            slice_sizes=(1,),
                            mode=lax.GatherScatterMode.PROMISE_IN_BOUNDS)
                        pos = base + u
                        v = rows[rs, slab, pos, pl.ds(0, _DH)]
                        v0, v1 = plsc.unpack(v, format=plsc.PackFormat.INTERLEAVED)
                        acc0 = acc0 + wj * v0
                        acc1 = acc1 + wj * v1
                outb[q, pl.ds(h * _DH, 16)] = acc0
                outb[q, pl.ds(h * _DH + 16, 16)] = acc1
                return carry

            lax.fori_loop(0, _QG, qstep, 0)

        @pl.when(i < ng - 1)
        def _():
            wait_iw((i + 1) & 1)
            issue_gathers((i + 1) & 1, 0, 0, sem_r0)

        @pl.when(i < ng - 2)
        def _():
            issue_iw(s, g + 32)

        pltpu.sync_copy(outb, out.at[b, pl.ds(g * _QG, _QG), pl.ds(0, _EMBED)])
        return carry

    lax.fori_loop(0, ng, group, 0)


def _make_sample():
    mesh = plsc.VectorSubcoreMesh(core_axis_name="c", subcore_axis_name="s")
    return functools.partial(
        pl.kernel,
        mesh=mesh,
        compiler_params=pltpu.CompilerParams(
            needs_layout_passes=False, use_tc_tiling_on_sc=False),
        out_type=jax.ShapeDtypeStruct((_BS, _NQ, _EMBED), jnp.float32),
        scratch_types=[
            pltpu.VMEM((2, _HEADS, 8, 128), jnp.int32),
            pltpu.VMEM((2, _HEADS, 8, 128), jnp.float32),
            pltpu.VMEM((2, 8, 128, _DH), jnp.bfloat16),
            pltpu.VMEM((_QG, _EMBED), jnp.float32),
            pltpu.SemaphoreType.DMA,
            pltpu.SemaphoreType.DMA,
            pltpu.SemaphoreType.DMA,
        ],
    )(_sample_body)


_E_NP = np.zeros((_LEVELS, _HLP), np.float32)
for _j in range(_HLP):
    _E_NP[(_j >> 2) & 3, _j] = 1.0
_G_NP = np.zeros((_HLP, _HLP), np.float32)
for _k in range(_HLP):
    for _j in range(_HLP):
        if (_k >> 4) == (_j >> 4):
            _G_NP[_k, _j] = 1.0


def kernel(query, reference_points, spatial_shapes, level_start_index,
           W_off, b_off, W_attn, b_attn, W_val, b_val, W_out, b_out):
    del spatial_shapes, level_start_index
    rx = reference_points[..., 0]
    ry = reference_points[..., 1]
    wox = W_off.reshape(_EMBED, _HLP, 2)[..., 0]
    woy = W_off.reshape(_EMBED, _HLP, 2)[..., 1]
    box = b_off.reshape(_HLP, 2)[:, 0].reshape(1, _HLP)
    boy = b_off.reshape(_HLP, 2)[:, 1].reshape(1, _HLP)
    perm = np.zeros(_EMBED, np.int32)
    for h_ in range(_HEADS):
        for m_ in range(16):
            perm[h_ * _DH + 2 * m_] = h_ * _DH + m_
            perm[h_ * _DH + 2 * m_ + 1] = h_ * _DH + 16 + m_
    wval_p = W_val[:, perm]
    bval = b_val[perm].reshape(1, _EMBED)
    battn = b_attn.reshape(1, _HLP)
    bout = b_out.reshape(1, _EMBED)
    e_mat = jnp.asarray(_E_NP)
    g_mat = jnp.asarray(_G_NP)

    full = lambda shape: pl.BlockSpec(shape, lambda b_, q_: tuple(0 for _ in shape))
    val, idx, wgt = pl.pallas_call(
        _prep_body,
        grid=(_BS, _NQB),
        in_specs=[
            pl.BlockSpec((1, _QB, _EMBED), lambda b_, q_: (b_, q_, 0)),
            pl.BlockSpec((1, _QB, _LEVELS), lambda b_, q_: (b_, q_, 0)),
            pl.BlockSpec((1, _QB, _LEVELS), lambda b_, q_: (b_, q_, 0)),
            full((_EMBED, _EMBED)),
            full((1, _EMBED)),
            full((_EMBED, _HLP)),
            full((1, _HLP)),
            full((_EMBED, _HLP)),
            full((1, _HLP)),
            full((_EMBED, _HLP)),
            full((1, _HLP)),
            full((_LEVELS, _HLP)),
            full((_HLP, _HLP)),
        ],
        out_specs=[
            pl.BlockSpec((1, _QB, _EMBED), lambda b_, q_: (b_, q_, 0)),
            pl.BlockSpec((_HEADS, _QB, 64), lambda b_, q_: (0, b_ * _NQB + q_, 0)),
            pl.BlockSpec((_HEADS, _QB, 64), lambda b_, q_: (0, b_ * _NQB + q_, 0)),
        ],
        out_shape=[
            jax.ShapeDtypeStruct((_BS, _NQ, _EMBED), jnp.bfloat16),
            jax.ShapeDtypeStruct((_HEADS, _BS * _NQ, 64), jnp.int32),
            jax.ShapeDtypeStruct((_HEADS, _BS * _NQ, 64), jnp.float32),
        ],
    )(query, rx, ry, wval_p, bval, wox, box, woy, boy, W_attn, battn, e_mat, g_mat)

    vtab = val.reshape(_BS * _NQ * _HEADS, _DH)
    idx5 = idx.reshape(_HEADS, _BS, _NQ // _QG, 8, 128)
    wgt5 = wgt.reshape(_HEADS, _BS, _NQ // _QG, 8, 128)
    sampled = _make_sample()(vtab, idx5, wgt5)

    out = pl.pallas_call(
        _outp_body,
        grid=(_BS, _NQB),
        in_specs=[
            pl.BlockSpec((1, _QB, _EMBED), lambda b_, q_: (b_, q_, 0)),
            pl.BlockSpec((1, _QB, _EMBED), lambda b_, q_: (b_, q_, 0)),
            full((_EMBED, _EMBED)),
            full((1, _EMBED)),
        ],
        out_specs=pl.BlockSpec((1, _QB, _EMBED), lambda b_, q_: (b_, q_, 0)),
        out_shape=jax.ShapeDtypeStruct((_BS, _NQ, _EMBED), jnp.float32),
    )(sampled, query, W_out, bout)
    return out


# speedup vs baseline: 1.5054x; 1.0264x over previous
import functools
import math

import jax
import jax.numpy as jnp
import numpy as np
from jax import lax
from jax.experimental import pallas as pl
from jax.experimental.pallas import tpu as pltpu
from jax.experimental.pallas import tpu_sc as plsc

_BS = 2
_NQ = 5440
_EMBED = 256
_HEADS = 8
_LEVELS = 4
_POINTS = 4
_DH = _EMBED // _HEADS
_HLP = _HEADS * _LEVELS * _POINTS
_QB = 544
_NQB = _NQ // _QB

_SIZES = [64 >> l for l in range(_LEVELS)]
_BASES = [0, 4096, 5120, 5376]


def _prep_body(q_ref, rx_ref, ry_ref, wval_ref, bval_ref, wox_ref, box_ref,
               woy_ref, boy_ref, wattn_ref, battn_ref, e_ref, g_ref,
               val_ref, idx_ref, wgt_ref):
    b = pl.program_id(0)
    q = q_ref[0]
    val_ref[0] = (jnp.dot(q, wval_ref[...], preferred_element_type=jnp.float32, precision=jax.lax.Precision.HIGHEST)
                  + bval_ref[...]).astype(jnp.bfloat16)

    offx = jnp.dot(q, wox_ref[...], preferred_element_type=jnp.float32, precision=jax.lax.Precision.HIGHEST) + box_ref[...]
    offy = jnp.dot(q, woy_ref[...], preferred_element_type=jnp.float32, precision=jax.lax.Precision.HIGHEST) + boy_ref[...]
    aw = jnp.dot(q, wattn_ref[...], preferred_element_type=jnp.float32, precision=jax.lax.Precision.HIGHEST) + battn_ref[...]
    m = jnp.max(aw, axis=-1, keepdims=True)
    ex = jnp.exp(aw - m)
    a = ex / jnp.dot(ex, g_ref[...], preferred_element_type=jnp.float32, precision=jax.lax.Precision.HIGHEST)

    col = lax.broadcasted_iota(jnp.int32, (1, _HLP), 1)
    lvl = (col >> 2) & 3
    head = col >> 4
    wi = jnp.right_shift(jnp.int32(64), lvl)
    wf = wi.astype(jnp.float32)
    base = jnp.where(lvl == 0, 0,
                     jnp.where(lvl == 1, _BASES[1],
                               jnp.where(lvl == 2, _BASES[2], _BASES[3])))

    rx = jnp.dot(rx_ref[0], e_ref[...], preferred_element_type=jnp.float32, precision=jax.lax.Precision.HIGHEST)
    ry = jnp.dot(ry_ref[0], e_ref[...], preferred_element_type=jnp.float32, precision=jax.lax.Precision.HIGHEST)
    x = rx * wf + offx - 0.5
    y = ry * wf + offy - 0.5

    x0f = jnp.floor(x)
    y0f = jnp.floor(y)
    fx = x - x0f
    fy = y - y0f
    x0 = x0f.astype(jnp.int32)
    y0 = y0f.astype(jnp.int32)
    x1 = x0 + 1
    y1 = y0 + 1
    vx0 = (x0 >= 0) & (x0 < wi)
    vx1 = (x1 >= 0) & (x1 < wi)
    vy0 = (y0 >= 0) & (y0 < wi)
    vy1 = (y1 >= 0) & (y1 < wi)
    xc0 = jnp.clip(x0, 0, wi - 1)
    xc1 = jnp.clip(x1, 0, wi - 1)
    yc0 = jnp.clip(y0, 0, wi - 1)
    yc1 = jnp.clip(y1, 0, wi - 1)

    qoff = (b * _NQ + base) * _HEADS + head

    def flat(yc, xc):
        return qoff + (yc * wi + xc) * _HEADS

    gx0 = 1.0 - fx
    gy0 = 1.0 - fy
    i00 = flat(yc0, xc0)
    i10 = flat(yc0, xc1)
    i01 = flat(yc1, xc0)
    i11 = flat(yc1, xc1)
    w00 = gx0 * gy0 * (vx0 & vy0).astype(jnp.float32) * a
    w10 = fx * gy0 * (vx1 & vy0).astype(jnp.float32) * a
    w01 = gx0 * fy * (vx0 & vy1).astype(jnp.float32) * a
    w11 = fx * fy * (vx1 & vy1).astype(jnp.float32) * a

    for hh in range(_HEADS):
        sl = slice(hh * 16, hh * 16 + 16)
        idx_ref[hh] = jnp.concatenate(
            [i00[:, sl], i10[:, sl], i01[:, sl], i11[:, sl]], axis=-1)
        wgt_ref[hh] = jnp.concatenate(
            [w00[:, sl], w10[:, sl], w01[:, sl], w11[:, sl]], axis=-1)


def _outp_body(s_ref, q_ref, wout_ref, bout_ref, o_ref):
    o_ref[0] = (jnp.dot(s_ref[0], wout_ref[...], preferred_element_type=jnp.float32, precision=jax.lax.Precision.HIGHEST)
                + bout_ref[...] + q_ref[0])


_NC = 2
_NS = 16
_QG = 16
_NG = (_NQ // 2) // _QG


_NGT = _NQ // _QG


def _sample_body(vtab, idx_hbm, wgt_hbm, out, idxb, wgtb, rows, outb,
                 sem_iw, sem_r0, sem_r1, sem_r2, sem_r3):
    wid = lax.axis_index("s") * _NC + lax.axis_index("c")
    b = wid >> 4
    t = wid & 15
    ng = (_NGT - t + 15) // 16
    lane = lax.iota(jnp.int32, _QG)
    jvec = lane >> 1
    pbase = (lane & 1) * 64

    def issue_iw(gslot, g):
        pltpu.async_copy(idx_hbm.at[pl.ds(0, _HEADS), b, g], idxb.at[gslot], sem_iw)
        pltpu.async_copy(wgt_hbm.at[pl.ds(0, _HEADS), b, g], wgtb.at[gslot], sem_iw)

    def wait_iw(gslot):
        pltpu.make_async_copy(
            idx_hbm.at[pl.ds(0, _HEADS), b, 0], idxb.at[gslot], sem_iw).wait()
        pltpu.make_async_copy(
            wgt_hbm.at[pl.ds(0, _HEADS), b, 0], wgtb.at[gslot], sem_iw).wait()

    def issue_gathers(gslot, h, rslot, sem):
        for j in range(8):
            pltpu.async_copy(vtab.at[idxb.at[gslot, h, j]], rows.at[rslot, j], sem)

    def wait_gathers(gslot, h, rslot, sem):
        for j in range(8):
            pltpu.make_async_copy(
                vtab.at[idxb.at[gslot, h, j]], rows.at[rslot, j], sem).wait()

    sems = [sem_r0, sem_r1, sem_r2, sem_r3]
    issue_iw(0, t)
    wait_iw(0)
    issue_gathers(0, 0, 0, sem_r0)
    issue_gathers(0, 1, 1, sem_r1)
    issue_iw(1, t + 16)

    def group(i, carry):
        s = i & 1
        g = t + i * 16
        for h in range(_HEADS):
            rs = h & 3
            wait_gathers(s, h, rs, sems[rs])
            if h < _HEADS - 2:
                nrs = (h + 2) & 3
                issue_gathers(s, h + 2, nrs, sems[nrs])
            elif h == _HEADS - 2:
                @pl.when(i < ng - 1)
                def _():
                    wait_iw((i + 1) & 1)
                    issue_gathers((i + 1) & 1, 0, 0, sems[0])
            else:
                @pl.when(i < ng - 1)
                def _():
                    issue_gathers((i + 1) & 1, 1, 1, sems[1])

            def qstep(q, carry, h=h, rs=rs, s=s):
                slab = q >> 1
                pb = (q & 1) * 64
                acc0 = jnp.zeros((_QG,), jnp.float32)
                acc1 = jnp.zeros((_QG,), jnp.float32)
                for k in range(4):
                    base = pb + k * 16
                    wvec = wgtb[s, h, slab, pl.ds(base, 16)]
                    for u in range(16):
                        wj = lax.gather(
                            wvec, jnp.full((_QG, 1), u, jnp.int32),
                            dimension_numbers=lax.GatherDimensionNumbers(
                                offset_dims=(), collapsed_slice_dims=(0,),
                                start_index_map=(0,)),
                            slice_sizes=(1,),
                            mode=lax.GatherScatterMode.PROMISE_IN_BOUNDS)
                        pos = base + u
                        v = rows[rs, slab, pos, pl.ds(0, _DH)]
                        v0, v1 = plsc.unpack(v, format=plsc.PackFormat.INTERLEAVED)
                        acc0 = acc0 + wj * v0
                        acc1 = acc1 + wj * v1
                outb[q, pl.ds(h * _DH, 16)] = acc0
                outb[q, pl.ds(h * _DH + 16, 16)] = acc1
                return carry

            lax.fori_loop(0, _QG, qstep, 0)

        @pl.when(i < ng - 2)
        def _():
            issue_iw(s, g + 32)

        pltpu.sync_copy(outb, out.at[b, pl.ds(g * _QG, _QG), pl.ds(0, _EMBED)])
        return carry

    lax.fori_loop(0, ng, group, 0)


def _make_sample():
    mesh = plsc.VectorSubcoreMesh(core_axis_name="c", subcore_axis_name="s")
    return functools.partial(
        pl.kernel,
        mesh=mesh,
        compiler_params=pltpu.CompilerParams(
            needs_layout_passes=False, use_tc_tiling_on_sc=False),
        out_type=jax.ShapeDtypeStruct((_BS, _NQ, _EMBED), jnp.float32),
        scratch_types=[
            pltpu.VMEM((2, _HEADS, 8, 128), jnp.int32),
            pltpu.VMEM((2, _HEADS, 8, 128), jnp.float32),
            pltpu.VMEM((4, 8, 128, _DH), jnp.bfloat16),
            pltpu.VMEM((_QG, _EMBED), jnp.float32),
            pltpu.SemaphoreType.DMA,
            pltpu.SemaphoreType.DMA,
            pltpu.SemaphoreType.DMA,
            pltpu.SemaphoreType.DMA,
            pltpu.SemaphoreType.DMA,
        ],
    )(_sample_body)


_E_NP = np.zeros((_LEVELS, _HLP), np.float32)
for _j in range(_HLP):
    _E_NP[(_j >> 2) & 3, _j] = 1.0
_G_NP = np.zeros((_HLP, _HLP), np.float32)
for _k in range(_HLP):
    for _j in range(_HLP):
        if (_k >> 4) == (_j >> 4):
            _G_NP[_k, _j] = 1.0


def kernel(query, reference_points, spatial_shapes, level_start_index,
           W_off, b_off, W_attn, b_attn, W_val, b_val, W_out, b_out):
    del spatial_shapes, level_start_index
    rx = reference_points[..., 0]
    ry = reference_points[..., 1]
    wox = W_off.reshape(_EMBED, _HLP, 2)[..., 0]
    woy = W_off.reshape(_EMBED, _HLP, 2)[..., 1]
    box = b_off.reshape(_HLP, 2)[:, 0].reshape(1, _HLP)
    boy = b_off.reshape(_HLP, 2)[:, 1].reshape(1, _HLP)
    perm = np.zeros(_EMBED, np.int32)
    for h_ in range(_HEADS):
        for m_ in range(16):
            perm[h_ * _DH + 2 * m_] = h_ * _DH + m_
            perm[h_ * _DH + 2 * m_ + 1] = h_ * _DH + 16 + m_
    wval_p = W_val[:, perm]
    bval = b_val[perm].reshape(1, _EMBED)
    battn = b_attn.reshape(1, _HLP)
    bout = b_out.reshape(1, _EMBED)
    e_mat = jnp.asarray(_E_NP)
    g_mat = jnp.asarray(_G_NP)

    full = lambda shape: pl.BlockSpec(shape, lambda b_, q_: tuple(0 for _ in shape))
    val, idx, wgt = pl.pallas_call(
        _prep_body,
        grid=(_BS, _NQB),
        in_specs=[
            pl.BlockSpec((1, _QB, _EMBED), lambda b_, q_: (b_, q_, 0)),
            pl.BlockSpec((1, _QB, _LEVELS), lambda b_, q_: (b_, q_, 0)),
            pl.BlockSpec((1, _QB, _LEVELS), lambda b_, q_: (b_, q_, 0)),
            full((_EMBED, _EMBED)),
            full((1, _EMBED)),
            full((_EMBED, _HLP)),
            full((1, _HLP)),
            full((_EMBED, _HLP)),
            full((1, _HLP)),
            full((_EMBED, _HLP)),
            full((1, _HLP)),
            full((_LEVELS, _HLP)),
            full((_HLP, _HLP)),
        ],
        out_specs=[
            pl.BlockSpec((1, _QB, _EMBED), lambda b_, q_: (b_, q_, 0)),
            pl.BlockSpec((_HEADS, _QB, 64), lambda b_, q_: (0, b_ * _NQB + q_, 0)),
            pl.BlockSpec((_HEADS, _QB, 64), lambda b_, q_: (0, b_ * _NQB + q_, 0)),
        ],
        out_shape=[
            jax.ShapeDtypeStruct((_BS, _NQ, _EMBED), jnp.bfloat16),
            jax.ShapeDtypeStruct((_HEADS, _BS * _NQ, 64), jnp.int32),
            jax.ShapeDtypeStruct((_HEADS, _BS * _NQ, 64), jnp.float32),
        ],
    )(query, rx, ry, wval_p, bval, wox, box, woy, boy, W_attn, battn, e_mat, g_mat)

    vtab = val.reshape(_BS * _NQ * _HEADS, _DH)
    idx5 = idx.reshape(_HEADS, _BS, _NQ // _QG, 8, 128)
    wgt5 = wgt.reshape(_HEADS, _BS, _NQ // _QG, 8, 128)
    sampled = _make_sample()(vtab, idx5, wgt5)

    out = pl.pallas_call(
        _outp_body,
        grid=(_BS, _NQB),
        in_specs=[
            pl.BlockSpec((1, _QB, _EMBED), lambda b_, q_: (b_, q_, 0)),
            pl.BlockSpec((1, _QB, _EMBED), lambda b_, q_: (b_, q_, 0)),
            full((_EMBED, _EMBED)),
            full((1, _EMBED)),
        ],
        out_specs=pl.BlockSpec((1, _QB, _EMBED), lambda b_, q_: (b_, q_, 0)),
        out_shape=jax.ShapeDtypeStruct((_BS, _NQ, _EMBED), jnp.float32),
    )(sampled, query, W_out, bout)
    return out
